# Initial kernel scaffold; baseline (speedup 1.0000x reference)
import jax
import jax.numpy as jnp
from jax.experimental import pallas as pl


def kernel(x, edge, W1l, b1l, W1r, W2l, b2l, W2r):
    raise NotImplementedError("write your pallas kernel here")


# speedup vs baseline: 3.6507x; 3.6507x over previous
import functools

import jax
import jax.numpy as jnp
from jax import lax
from jax.experimental import pallas as pl
from jax.experimental.pallas import tpu as pltpu
from jax.experimental.pallas import tpu_sc as plsc

N_NODES = 10000
N_EDGES = 320000
IN_DIM = 128
HIDDEN = 256
OUT_DIM = 128

NC = 2
NS = 16
NW = NC * NS
C = 128
E_PAD = 327680
CH = E_PAD // (NW * C)
N_PAD = 10240
RPS = N_PAD // NS


def _sc_agg_body(feat, src_h, dst_h, z128, acc_out,
                 src_v, dst_v, rows_v, acc_sh, sem):
    c = lax.axis_index("c")
    s = lax.axis_index("s")
    wid = c * NS + s

    pltpu.sync_copy(src_h.at[wid], src_v)
    pltpu.sync_copy(dst_h.at[wid], dst_v)
    pltpu.sync_copy(z128, rows_v)
    for k in range(RPS // C):
        pltpu.sync_copy(rows_v, acc_sh.at[pl.ds(s * RPS + k * C, C)])
    plsc.subcore_barrier()

    @pl.loop(0, CH)
    def _(j):
        pltpu.async_copy(feat.at[src_v.at[j]], rows_v, sem).wait()
        pltpu.sync_copy(rows_v, acc_sh.at[dst_v.at[j]], add=True)

    plsc.subcore_barrier()
    for k in range(RPS // C):
        pltpu.sync_copy(acc_sh.at[pl.ds(s * RPS + k * C, C)], rows_v)
        pltpu.sync_copy(rows_v, acc_out.at[c, pl.ds(s * RPS + k * C, C)])


def _make_sc_agg():
    mesh = plsc.VectorSubcoreMesh(core_axis_name="c", subcore_axis_name="s")
    return pl.kernel(
        _sc_agg_body,
        out_type=[jax.ShapeDtypeStruct((NC, N_PAD, 128), jnp.float32)],
        mesh=mesh,
        scratch_types=[
            pltpu.VMEM((CH, C), jnp.int32),
            pltpu.VMEM((CH, C), jnp.int32),
            pltpu.VMEM((C, 128), jnp.float32),
            pltpu.VMEM_SHARED((N_PAD, 128), jnp.float32),
            pltpu.SemaphoreType.DMA,
        ],
    )


def _sc_deg_body(dst_h, z128, ones_h, deg_out, dst_v, ones_v, zb_v, deg_sh):
    c = lax.axis_index("c")
    s = lax.axis_index("s")
    wid = c * NS + s

    pltpu.sync_copy(dst_h.at[wid], dst_v)
    pltpu.sync_copy(ones_h, ones_v)
    pltpu.sync_copy(z128, zb_v)
    for k in range(RPS // C):
        pltpu.sync_copy(zb_v, deg_sh.at[pl.ds(s * RPS + k * C, C)])
    plsc.subcore_barrier()

    @pl.loop(0, CH)
    def _(j):
        pltpu.sync_copy(ones_v, deg_sh.at[dst_v.at[j]], add=True)

    plsc.subcore_barrier()
    for k in range(RPS // C):
        pltpu.sync_copy(deg_sh.at[pl.ds(s * RPS + k * C, C)], zb_v)
        pltpu.sync_copy(zb_v, deg_out.at[c, pl.ds(s * RPS + k * C, C)])


def _make_sc_deg():
    mesh = plsc.VectorSubcoreMesh(core_axis_name="c", subcore_axis_name="s")
    return pl.kernel(
        _sc_deg_body,
        out_type=[jax.ShapeDtypeStruct((NC, N_PAD, 128), jnp.float32)],
        mesh=mesh,
        scratch_types=[
            pltpu.VMEM((CH, C), jnp.int32),
            pltpu.VMEM((C, 128), jnp.float32),
            pltpu.VMEM((C, 128), jnp.float32),
            pltpu.VMEM_SHARED((N_PAD, 128), jnp.float32),
        ],
    )


def _tc1_body(agg, deg, x, w1l, b1l, w1r, w2l, h_out, hl_out):
    dd = deg[0][:, 0:1] + deg[1][:, 0:1]
    inv = 1.0 / jnp.maximum(dd, 1.0)
    mean = (agg[0] + agg[1]) * inv
    dn = (((1,), (1,)), ((), ()))
    pre = lax.dot_general(mean, w1l[...], dn,
                          preferred_element_type=jnp.float32,
                          precision=lax.Precision.HIGHEST)
    pre = pre + lax.dot_general(x[...], w1r[...], dn,
                                preferred_element_type=jnp.float32,
                                precision=lax.Precision.HIGHEST)
    h = jnp.maximum(pre + b1l[...], 0.0)
    h_out[...] = h
    hl_out[...] = lax.dot_general(h, w2l[...], dn,
                                  preferred_element_type=jnp.float32,
                                  precision=lax.Precision.HIGHEST)


def _tc2_body(agg, deg, h, w2r, b2l, out):
    dd = deg[0][:, 0:1] + deg[1][:, 0:1]
    inv = 1.0 / jnp.maximum(dd, 1.0)
    mean = (agg[0] + agg[1]) * inv
    dn = (((1,), (1,)), ((), ()))
    out[...] = mean + b2l[...] + lax.dot_general(
        h[...], w2r[...], dn,
        preferred_element_type=jnp.float32,
        precision=lax.Precision.HIGHEST)


_R = 400
_GRID = N_NODES // _R


def _tc1(agg1, deg, x, W1l, b1l, W1r, W2l):
    return pl.pallas_call(
        _tc1_body,
        grid=(_GRID,),
        in_specs=[
            pl.BlockSpec((NC, _R, 128), lambda i: (0, i, 0)),
            pl.BlockSpec((NC, _R, 128), lambda i: (0, i, 0)),
            pl.BlockSpec((_R, IN_DIM), lambda i: (i, 0)),
            pl.BlockSpec((HIDDEN, IN_DIM), lambda i: (0, 0)),
            pl.BlockSpec((1, HIDDEN), lambda i: (0, 0)),
            pl.BlockSpec((HIDDEN, IN_DIM), lambda i: (0, 0)),
            pl.BlockSpec((OUT_DIM, HIDDEN), lambda i: (0, 0)),
        ],
        out_specs=[
            pl.BlockSpec((_R, HIDDEN), lambda i: (i, 0)),
            pl.BlockSpec((_R, OUT_DIM), lambda i: (i, 0)),
        ],
        out_shape=[
            jax.ShapeDtypeStruct((N_NODES, HIDDEN), jnp.float32),
            jax.ShapeDtypeStruct((N_NODES, OUT_DIM), jnp.float32),
        ],
    )(agg1, deg, x, W1l, b1l, W1r, W2l)


def _tc2(agg2, deg, h, W2r, b2l):
    return pl.pallas_call(
        _tc2_body,
        grid=(_GRID,),
        in_specs=[
            pl.BlockSpec((NC, _R, 128), lambda i: (0, i, 0)),
            pl.BlockSpec((NC, _R, 128), lambda i: (0, i, 0)),
            pl.BlockSpec((_R, HIDDEN), lambda i: (i, 0)),
            pl.BlockSpec((OUT_DIM, HIDDEN), lambda i: (0, 0)),
            pl.BlockSpec((1, OUT_DIM), lambda i: (0, 0)),
        ],
        out_specs=pl.BlockSpec((_R, OUT_DIM), lambda i: (i, 0)),
        out_shape=jax.ShapeDtypeStruct((N_NODES, OUT_DIM), jnp.float32),
    )(agg2, deg, h, W2r, b2l)


@jax.jit
def kernel(x, edge, W1l, b1l, W1r, W2l, b2l, W2r):
    edge = edge.astype(jnp.int32)
    npad = E_PAD - N_EDGES
    src = jnp.concatenate([edge[0], jnp.zeros((npad,), jnp.int32)])
    dst = jnp.concatenate(
        [edge[1], N_NODES + (jnp.arange(npad, dtype=jnp.int32) % (N_PAD - N_NODES))])
    src3 = src.reshape(NW, CH, C)
    dst3 = dst.reshape(NW, CH, C)

    z128 = jnp.zeros((C, 128), jnp.float32)
    ones128 = jnp.ones((C, 128), jnp.float32)

    (deg,) = _make_sc_deg()(dst3, z128, ones128)
    (agg1,) = _make_sc_agg()(x, src3, dst3, z128)
    h, hl = _tc1(agg1, deg, x, W1l, b1l.reshape(1, HIDDEN), W1r, W2l)
    (agg2,) = _make_sc_agg()(hl, src3, dst3, z128)
    out = _tc2(agg2, deg, h, W2r, b2l.reshape(1, OUT_DIM))
    return out


# speedup vs baseline: 3.9469x; 1.0811x over previous
import functools

import jax
import jax.numpy as jnp
from jax import lax
from jax.experimental import pallas as pl
from jax.experimental.pallas import tpu as pltpu
from jax.experimental.pallas import tpu_sc as plsc

N_NODES = 10000
N_EDGES = 320000
IN_DIM = 128
HIDDEN = 256
OUT_DIM = 128

NC = 2
NS = 16
NW = NC * NS
C = 128
E_PAD = 327680
CH = E_PAD // (NW * C)
N_PAD = 10240
RPS = N_PAD // NS


def _sc_agg_body(feat, src_h, dst_h, z128, acc_out,
                 rows_v, rows_b, ix_a, ix_b, dx_a, dx_b,
                 acc_sh, sem, sem_b):
    c = lax.axis_index("c")
    s = lax.axis_index("s")
    wid = c * NS + s

    pltpu.sync_copy(z128, rows_v)
    for k in range(RPS // C):
        pltpu.sync_copy(rows_v, acc_sh.at[pl.ds(s * RPS + k * C, C)])
    plsc.subcore_barrier()

    pltpu.sync_copy(src_h.at[wid, 0], ix_a)
    pltpu.async_copy(feat.at[ix_a], rows_v, sem)

    @pl.loop(0, CH, step=2)
    def _(j):
        pltpu.sync_copy(src_h.at[wid, j + 1], ix_b)
        pltpu.async_copy(feat.at[ix_b], rows_b, sem_b)
        pltpu.sync_copy(dst_h.at[wid, j], dx_a)
        pltpu.make_async_copy(feat.at[ix_a], rows_v, sem).wait()
        pltpu.sync_copy(rows_v, acc_sh.at[dx_a], add=True)

        @pl.when(j + 2 < CH)
        def _():
            pltpu.sync_copy(src_h.at[wid, j + 2], ix_a)
            pltpu.async_copy(feat.at[ix_a], rows_v, sem)

        pltpu.sync_copy(dst_h.at[wid, j + 1], dx_b)
        pltpu.make_async_copy(feat.at[ix_b], rows_b, sem_b).wait()
        pltpu.sync_copy(rows_b, acc_sh.at[dx_b], add=True)

    plsc.subcore_barrier()
    for k in range(RPS // C):
        pltpu.sync_copy(acc_sh.at[pl.ds(s * RPS + k * C, C)], rows_v)
        pltpu.sync_copy(rows_v, acc_out.at[c, pl.ds(s * RPS + k * C, C)])


def _make_sc_agg():
    mesh = plsc.VectorSubcoreMesh(core_axis_name="c", subcore_axis_name="s")
    return pl.kernel(
        _sc_agg_body,
        out_type=[jax.ShapeDtypeStruct((NC, N_PAD, 128), jnp.float32)],
        mesh=mesh,
        scratch_types=[
            pltpu.VMEM((C, 128), jnp.float32),
            pltpu.VMEM((C, 128), jnp.float32),
            pltpu.VMEM((C,), jnp.int32),
            pltpu.VMEM((C,), jnp.int32),
            pltpu.VMEM((C,), jnp.int32),
            pltpu.VMEM((C,), jnp.int32),
            pltpu.VMEM_SHARED((N_PAD, 128), jnp.float32),
            pltpu.SemaphoreType.DMA,
            pltpu.SemaphoreType.DMA,
        ],
    )


def _sc_deg_body(dst_h, z128, ones_h, deg_out, dst_v, ones_v, zb_v, deg_sh):
    c = lax.axis_index("c")
    s = lax.axis_index("s")
    wid = c * NS + s

    pltpu.sync_copy(dst_h.at[wid], dst_v)
    pltpu.sync_copy(ones_h, ones_v)
    pltpu.sync_copy(z128, zb_v)
    for k in range(RPS // C):
        pltpu.sync_copy(zb_v, deg_sh.at[pl.ds(s * RPS + k * C, C)])
    plsc.subcore_barrier()

    @pl.loop(0, CH)
    def _(j):
        pltpu.sync_copy(ones_v, deg_sh.at[dst_v.at[j]], add=True)

    plsc.subcore_barrier()
    for k in range(RPS // C):
        pltpu.sync_copy(deg_sh.at[pl.ds(s * RPS + k * C, C)], zb_v)
        pltpu.sync_copy(zb_v, deg_out.at[c, pl.ds(s * RPS + k * C, C)])


def _make_sc_deg():
    mesh = plsc.VectorSubcoreMesh(core_axis_name="c", subcore_axis_name="s")
    return pl.kernel(
        _sc_deg_body,
        out_type=[jax.ShapeDtypeStruct((NC, N_PAD, 128), jnp.float32)],
        mesh=mesh,
        scratch_types=[
            pltpu.VMEM((CH, C), jnp.int32),
            pltpu.VMEM((C, 128), jnp.float32),
            pltpu.VMEM((C, 128), jnp.float32),
            pltpu.VMEM_SHARED((N_PAD, 128), jnp.float32),
        ],
    )


def _tc1_body(agg, deg, x, w1l, b1l, w1r, w2l, h_out, hl_out):
    dd = deg[0][:, 0:1] + deg[1][:, 0:1]
    inv = 1.0 / jnp.maximum(dd, 1.0)
    mean = (agg[0] + agg[1]) * inv
    dn = (((1,), (1,)), ((), ()))
    pre = lax.dot_general(mean, w1l[...], dn,
                          preferred_element_type=jnp.float32,
                          precision=lax.Precision.HIGHEST)
    pre = pre + lax.dot_general(x[...], w1r[...], dn,
                                preferred_element_type=jnp.float32,
                                precision=lax.Precision.HIGHEST)
    h = jnp.maximum(pre + b1l[...], 0.0)
    h_out[...] = h
    hl_out[...] = lax.dot_general(h, w2l[...], dn,
                                  preferred_element_type=jnp.float32,
                                  precision=lax.Precision.HIGHEST)


def _tc2_body(agg, deg, h, w2r, b2l, out):
    dd = deg[0][:, 0:1] + deg[1][:, 0:1]
    inv = 1.0 / jnp.maximum(dd, 1.0)
    mean = (agg[0] + agg[1]) * inv
    dn = (((1,), (1,)), ((), ()))
    out[...] = mean + b2l[...] + lax.dot_general(
        h[...], w2r[...], dn,
        preferred_element_type=jnp.float32,
        precision=lax.Precision.HIGHEST)


_R = 400
_GRID = N_NODES // _R


def _tc1(agg1, deg, x, W1l, b1l, W1r, W2l):
    return pl.pallas_call(
        _tc1_body,
        grid=(_GRID,),
        in_specs=[
            pl.BlockSpec((NC, _R, 128), lambda i: (0, i, 0)),
            pl.BlockSpec((NC, _R, 128), lambda i: (0, i, 0)),
            pl.BlockSpec((_R, IN_DIM), lambda i: (i, 0)),
            pl.BlockSpec((HIDDEN, IN_DIM), lambda i: (0, 0)),
            pl.BlockSpec((1, HIDDEN), lambda i: (0, 0)),
            pl.BlockSpec((HIDDEN, IN_DIM), lambda i: (0, 0)),
            pl.BlockSpec((OUT_DIM, HIDDEN), lambda i: (0, 0)),
        ],
        out_specs=[
            pl.BlockSpec((_R, HIDDEN), lambda i: (i, 0)),
            pl.BlockSpec((_R, OUT_DIM), lambda i: (i, 0)),
        ],
        out_shape=[
            jax.ShapeDtypeStruct((N_NODES, HIDDEN), jnp.float32),
            jax.ShapeDtypeStruct((N_NODES, OUT_DIM), jnp.float32),
        ],
    )(agg1, deg, x, W1l, b1l, W1r, W2l)


def _tc2(agg2, deg, h, W2r, b2l):
    return pl.pallas_call(
        _tc2_body,
        grid=(_GRID,),
        in_specs=[
            pl.BlockSpec((NC, _R, 128), lambda i: (0, i, 0)),
            pl.BlockSpec((NC, _R, 128), lambda i: (0, i, 0)),
            pl.BlockSpec((_R, HIDDEN), lambda i: (i, 0)),
            pl.BlockSpec((OUT_DIM, HIDDEN), lambda i: (0, 0)),
            pl.BlockSpec((1, OUT_DIM), lambda i: (0, 0)),
        ],
        out_specs=pl.BlockSpec((_R, OUT_DIM), lambda i: (i, 0)),
        out_shape=jax.ShapeDtypeStruct((N_NODES, OUT_DIM), jnp.float32),
    )(agg2, deg, h, W2r, b2l)


@jax.jit
def kernel(x, edge, W1l, b1l, W1r, W2l, b2l, W2r):
    edge = edge.astype(jnp.int32)
    npad = E_PAD - N_EDGES
    src = jnp.concatenate([edge[0], jnp.zeros((npad,), jnp.int32)])
    dst = jnp.concatenate(
        [edge[1], N_NODES + (jnp.arange(npad, dtype=jnp.int32) % (N_PAD - N_NODES))])
    src3 = src.reshape(NW, CH, C)
    dst3 = dst.reshape(NW, CH, C)

    z128 = jnp.zeros((C, 128), jnp.float32)
    ones128 = jnp.ones((C, 128), jnp.float32)

    (deg,) = _make_sc_deg()(dst3, z128, ones128)
    (agg1,) = _make_sc_agg()(x, src3, dst3, z128)
    h, hl = _tc1(agg1, deg, x, W1l, b1l.reshape(1, HIDDEN), W1r, W2l)
    (agg2,) = _make_sc_agg()(hl, src3, dst3, z128)
    out = _tc2(agg2, deg, h, W2r, b2l.reshape(1, OUT_DIM))
    return out


# speedup vs baseline: 7.9384x; 2.0113x over previous
import functools

import jax
import jax.numpy as jnp
from jax import lax
from jax.experimental import pallas as pl
from jax.experimental.pallas import tpu as pltpu
from jax.experimental.pallas import tpu_sc as plsc

N_NODES = 10000
N_EDGES = 320000
IN_DIM = 128
HIDDEN = 256
OUT_DIM = 128

NC = 2
NS = 16
NW = NC * NS
C = 80
CH = N_EDGES // (NW * C)
N_PAD = 10240
RPS = N_PAD // NS


def _sc_agg_body(feat, src_h, dst_h, z128, acc_out,
                 rows_v, rows_b, ix_a, ix_b, dx_a, dx_b,
                 acc_sh, sem, sem_b):
    c = lax.axis_index("c")
    s = lax.axis_index("s")
    wid = c * NS + s

    pltpu.sync_copy(z128, rows_v)
    for k in range(RPS // C):
        pltpu.sync_copy(rows_v, acc_sh.at[pl.ds(s * RPS + k * C, C)])
    plsc.subcore_barrier()

    pltpu.sync_copy(src_h.at[wid, 0], ix_a)
    pltpu.async_copy(feat.at[ix_a], rows_v, sem)

    @pl.loop(0, CH, step=2)
    def _(j):
        pltpu.sync_copy(src_h.at[wid, j + 1], ix_b)
        pltpu.async_copy(feat.at[ix_b], rows_b, sem_b)
        pltpu.sync_copy(dst_h.at[wid, j], dx_a)
        pltpu.make_async_copy(feat.at[ix_a], rows_v, sem).wait()
        pltpu.sync_copy(rows_v, acc_sh.at[dx_a], add=True)

        @pl.when(j + 2 < CH)
        def _():
            pltpu.sync_copy(src_h.at[wid, j + 2], ix_a)
            pltpu.async_copy(feat.at[ix_a], rows_v, sem)

        pltpu.sync_copy(dst_h.at[wid, j + 1], dx_b)
        pltpu.make_async_copy(feat.at[ix_b], rows_b, sem_b).wait()
        pltpu.sync_copy(rows_b, acc_sh.at[dx_b], add=True)

    plsc.subcore_barrier()
    for k in range(RPS // C):
        pltpu.sync_copy(acc_sh.at[pl.ds(s * RPS + k * C, C)], rows_v)
        pltpu.sync_copy(rows_v, acc_out.at[c, pl.ds(s * RPS + k * C, C)])


def _make_sc_agg():
    mesh = plsc.VectorSubcoreMesh(core_axis_name="c", subcore_axis_name="s")
    return pl.kernel(
        _sc_agg_body,
        out_type=[jax.ShapeDtypeStruct((NC, N_PAD, 128), jnp.float32)],
        mesh=mesh,
        scratch_types=[
            pltpu.VMEM((C, 128), jnp.float32),
            pltpu.VMEM((C, 128), jnp.float32),
            pltpu.VMEM((C,), jnp.int32),
            pltpu.VMEM((C,), jnp.int32),
            pltpu.VMEM((C,), jnp.int32),
            pltpu.VMEM((C,), jnp.int32),
            pltpu.VMEM_SHARED((N_PAD, 128), jnp.float32),
            pltpu.SemaphoreType.DMA,
            pltpu.SemaphoreType.DMA,
        ],
    )


def _sc_deg_body(dst_h, z128, ones_h, deg_out, dst_v, ones_v, zb_v, deg_sh):
    c = lax.axis_index("c")
    s = lax.axis_index("s")
    wid = c * NS + s

    pltpu.sync_copy(dst_h.at[wid], dst_v)
    pltpu.sync_copy(ones_h, ones_v)
    pltpu.sync_copy(z128, zb_v)
    for k in range(RPS // C):
        pltpu.sync_copy(zb_v, deg_sh.at[pl.ds(s * RPS + k * C, C)])
    plsc.subcore_barrier()

    @pl.loop(0, CH)
    def _(j):
        pltpu.sync_copy(ones_v, deg_sh.at[dst_v.at[j]], add=True)

    plsc.subcore_barrier()
    for k in range(RPS // C):
        pltpu.sync_copy(deg_sh.at[pl.ds(s * RPS + k * C, C)], zb_v)
        pltpu.sync_copy(zb_v, deg_out.at[c, pl.ds(s * RPS + k * C, C)])


def _make_sc_deg():
    mesh = plsc.VectorSubcoreMesh(core_axis_name="c", subcore_axis_name="s")
    return pl.kernel(
        _sc_deg_body,
        out_type=[jax.ShapeDtypeStruct((NC, N_PAD, 128), jnp.float32)],
        mesh=mesh,
        scratch_types=[
            pltpu.VMEM((CH, C), jnp.int32),
            pltpu.VMEM((C, 128), jnp.float32),
            pltpu.VMEM((C, 128), jnp.float32),
            pltpu.VMEM_SHARED((N_PAD, 128), jnp.float32),
        ],
    )


def _tc1_body(agg, deg, x, w1l, b1l, w1r, w2l, h_out, hl_out):
    dd = deg[0][:, 0:1] + deg[1][:, 0:1]
    inv = 1.0 / jnp.maximum(dd, 1.0)
    mean = (agg[0] + agg[1]) * inv
    dn = (((1,), (1,)), ((), ()))
    pre = lax.dot_general(mean, w1l[...], dn,
                          preferred_element_type=jnp.float32,
                          precision=lax.Precision.HIGHEST)
    pre = pre + lax.dot_general(x[...], w1r[...], dn,
                                preferred_element_type=jnp.float32,
                                precision=lax.Precision.HIGHEST)
    h = jnp.maximum(pre + b1l[...], 0.0)
    h_out[...] = h
    hl_out[...] = lax.dot_general(h, w2l[...], dn,
                                  preferred_element_type=jnp.float32,
                                  precision=lax.Precision.HIGHEST)


def _tc2_body(agg, deg, h, w2r, b2l, out):
    dd = deg[0][:, 0:1] + deg[1][:, 0:1]
    inv = 1.0 / jnp.maximum(dd, 1.0)
    mean = (agg[0] + agg[1]) * inv
    dn = (((1,), (1,)), ((), ()))
    out[...] = mean + b2l[...] + lax.dot_general(
        h[...], w2r[...], dn,
        preferred_element_type=jnp.float32,
        precision=lax.Precision.HIGHEST)


_R = 400
_GRID = N_NODES // _R


def _tc1(agg1, deg, x, W1l, b1l, W1r, W2l):
    return pl.pallas_call(
        _tc1_body,
        grid=(_GRID,),
        in_specs=[
            pl.BlockSpec((NC, _R, 128), lambda i: (0, i, 0)),
            pl.BlockSpec((NC, _R, 128), lambda i: (0, i, 0)),
            pl.BlockSpec((_R, IN_DIM), lambda i: (i, 0)),
            pl.BlockSpec((HIDDEN, IN_DIM), lambda i: (0, 0)),
            pl.BlockSpec((1, HIDDEN), lambda i: (0, 0)),
            pl.BlockSpec((HIDDEN, IN_DIM), lambda i: (0, 0)),
            pl.BlockSpec((OUT_DIM, HIDDEN), lambda i: (0, 0)),
        ],
        out_specs=[
            pl.BlockSpec((_R, HIDDEN), lambda i: (i, 0)),
            pl.BlockSpec((_R, OUT_DIM), lambda i: (i, 0)),
        ],
        out_shape=[
            jax.ShapeDtypeStruct((N_NODES, HIDDEN), jnp.float32),
            jax.ShapeDtypeStruct((N_NODES, OUT_DIM), jnp.float32),
        ],
    )(agg1, deg, x, W1l, b1l, W1r, W2l)


def _tc2(agg2, deg, h, W2r, b2l):
    return pl.pallas_call(
        _tc2_body,
        grid=(_GRID,),
        in_specs=[
            pl.BlockSpec((NC, _R, 128), lambda i: (0, i, 0)),
            pl.BlockSpec((NC, _R, 128), lambda i: (0, i, 0)),
            pl.BlockSpec((_R, HIDDEN), lambda i: (i, 0)),
            pl.BlockSpec((OUT_DIM, HIDDEN), lambda i: (0, 0)),
            pl.BlockSpec((1, OUT_DIM), lambda i: (0, 0)),
        ],
        out_specs=pl.BlockSpec((_R, OUT_DIM), lambda i: (i, 0)),
        out_shape=jax.ShapeDtypeStruct((N_NODES, OUT_DIM), jnp.float32),
    )(agg2, deg, h, W2r, b2l)


@jax.jit
def kernel(x, edge, W1l, b1l, W1r, W2l, b2l, W2r):
    edge = edge.astype(jnp.int32)
    src3 = edge[0].reshape(NW, CH, C)
    dst3 = edge[1].reshape(NW, CH, C)

    z128 = jnp.zeros((C, 128), jnp.float32)
    ones128 = jnp.ones((C, 128), jnp.float32)

    (deg,) = _make_sc_deg()(dst3, z128, ones128)
    (agg1,) = _make_sc_agg()(x, src3, dst3, z128)
    h, hl = _tc1(agg1, deg, x, W1l, b1l.reshape(1, HIDDEN), W1r, W2l)
    (agg2,) = _make_sc_agg()(hl, src3, dst3, z128)
    out = _tc2(agg2, deg, h, W2r, b2l.reshape(1, OUT_DIM))
    return out
